# baseline (speedup 1.0000x reference)
import functools

import jax
import jax.numpy as jnp
from jax import lax
from jax.experimental import pallas as pl
from jax.experimental.pallas import tpu as pltpu
from jax.experimental.pallas import tpu_sc as plsc

VOCAB = 100000
DIM = 300
BATCH = 1024
MAX_NORM = 0.15


_NC, _NS = 2, 16
_NW = _NC * _NS
_B_PER_W = BATCH // _NW
_LANES = 16


def _sc_gather(table, idx):
    mesh = plsc.VectorSubcoreMesh(core_axis_name="c", subcore_axis_name="s")

    @functools.partial(
        pl.kernel,
        mesh=mesh,
        out_type=jax.ShapeDtypeStruct((BATCH, DIM), jnp.float32),
        scratch_types=[
            pltpu.VMEM((_B_PER_W,), jnp.int32),
            pltpu.VMEM((_B_PER_W, DIM), jnp.float32),
            pltpu.SemaphoreType.DMA,
        ],
    )
    def gather_kernel(table_hbm, idx_hbm, out_hbm, idx_v, rows_v, sem):
        wid = lax.axis_index("s") * _NC + lax.axis_index("c")
        base = wid * _B_PER_W
        pltpu.sync_copy(idx_hbm.at[pl.ds(base, _B_PER_W)], idx_v)
        copies = []
        for c in range(_B_PER_W // _LANES):
            chunk = idx_v[pl.ds(c * _LANES, _LANES)]
            for k in range(_LANES):
                j = c * _LANES + k
                row = chunk[k]
                cp = pltpu.make_async_copy(
                    table_hbm.at[pl.ds(row, 1)], rows_v.at[pl.ds(j, 1)], sem
                )
                cp.start()
                copies.append(cp)
        for cp in copies:
            cp.wait()
        pltpu.sync_copy(rows_v, out_hbm.at[pl.ds(base, _B_PER_W)])

    return gather_kernel(table, idx)


TILE_V = 2048
N_FULL = VOCAB // TILE_V
TAIL = VOCAB - N_FULL * TILE_V
N_STEPS = N_FULL + 1
RD = 3


def _w_read(w_ref, wb, rsem, step):
    return pltpu.make_async_copy(
        w_ref.at[pl.ds(step * TILE_V, TILE_V)], wb.at[step % RD], rsem.at[step % RD]
    )


def _w_read_tail(w_ref, wb, rsem):
    return pltpu.make_async_copy(
        w_ref.at[pl.ds(N_FULL * TILE_V, TAIL)],
        wb.at[N_FULL % RD].at[pl.ds(0, TAIL)],
        rsem.at[N_FULL % RD],
    )


def _mm_body(x_ref, w_ref, b_ref, out_ref, xs, wb, rsem):
    i = pl.program_id(0)

    @pl.when(i == 0)
    def _prologue():
        _w_read(w_ref, wb, rsem, 0).start()
        _w_read(w_ref, wb, rsem, 1).start()
        x = x_ref[...]
        nrm = jnp.sqrt(jnp.sum(x * x, axis=1, keepdims=True))
        scale = jnp.where(nrm > MAX_NORM, MAX_NORM / (nrm + 1e-7), 1.0)
        xs[...] = x * scale

    @pl.when(i + 2 < N_FULL)
    def _prefetch_w():
        _w_read(w_ref, wb, rsem, i + 2).start()

    @pl.when(i + 2 == N_FULL)
    def _prefetch_w_tail():
        _w_read_tail(w_ref, wb, rsem).start()

    @pl.when(i < N_FULL)
    def _wait_w():
        _w_read(w_ref, wb, rsem, i).wait()

    @pl.when(i == N_FULL)
    def _wait_w_tail():
        _w_read_tail(w_ref, wb, rsem).wait()

    acc = lax.dot_general(
        xs[...],
        wb[i % RD],
        (((1,), (1,)), ((), ())),
        preferred_element_type=jnp.float32,
    )
    out_ref[...] = acc + b_ref[...]


def _matmul(x, W, b2):
    return pl.pallas_call(
        _mm_body,
        grid=(N_STEPS,),
        in_specs=[
            pl.BlockSpec((BATCH, DIM), lambda i: (0, 0)),
            pl.BlockSpec(memory_space=pltpu.MemorySpace.HBM),
            pl.BlockSpec((1, TILE_V), lambda i: (0, i)),
        ],
        out_specs=pl.BlockSpec((BATCH, TILE_V), lambda i: (0, i)),
        out_shape=jax.ShapeDtypeStruct((BATCH, VOCAB), jnp.float32),
        scratch_shapes=[
            pltpu.VMEM((BATCH, DIM), jnp.float32),
            pltpu.VMEM((RD, TILE_V, DIM), jnp.float32),
            pltpu.SemaphoreType.DMA((RD,)),
        ],
    )(x, W, b2)


@jax.jit
def kernel(_inputs, target_table, W, b):
    idx = _inputs.astype(jnp.int32)
    x_raw = _sc_gather(target_table, idx)
    return _matmul(x_raw, W, b.reshape(1, VOCAB))
